# baseline (speedup 1.0000x reference)
import jax
import jax.numpy as jnp
from jax import lax
from jax.experimental import pallas as pl
from jax.experimental.pallas import tpu as pltpu
from jax.experimental.pallas import tpu_sc as plsc

VOCAB = 100000
D = 128
B_TOTAL = 4096 * 200
SCALE = float(D) ** 0.5

NC, NS, L = 2, 16, 16
NW = NC * NS
ROWS_PER_W = B_TOTAL // NW
CHUNK = 128
N_CHUNKS = ROWS_PER_W // CHUNK


def _body(idx_hbm, table_hbm, out_hbm,
          idx_v, g0, g1, s0, s1, semg0, semg1, sems0, sems1):
    cid = lax.axis_index("c")
    sid = lax.axis_index("s")
    wid = sid * NC + cid

    pltpu.sync_copy(idx_hbm.at[pl.ds(wid * N_CHUNKS, N_CHUNKS)], idx_v)

    out_base = pl.multiple_of(wid * ROWS_PER_W, CHUNK)

    def start_gather(c, buf, sem):
        pltpu.async_copy(table_hbm.at[idx_v.at[c]], buf, sem)

    def wait_gather(buf, sem):
        pltpu.make_async_copy(table_hbm.at[pl.ds(0, CHUNK)], buf, sem).wait()

    def start_scatter(c, buf, sem):
        pltpu.async_copy(buf, out_hbm.at[pl.ds(out_base + c * CHUNK, CHUNK)], sem)

    def wait_scatter(buf, sem):
        pltpu.make_async_copy(buf, out_hbm.at[pl.ds(0, CHUNK)], sem).wait()

    def scale_copy(src, dst):
        def row(r, _):
            for l in range(D // L):
                sl = pl.ds(l * L, L)
                dst[r, sl] = src[r, sl] * SCALE
            return 0
        lax.fori_loop(0, CHUNK, row, 0, unroll=4)

    start_gather(0, g0, semg0)
    start_gather(1, g1, semg1)

    def step(g, _):
        for b, (gb, sg, sb, ss) in enumerate(
                ((g0, semg0, s0, sems0), (g1, semg1, s1, sems1))):
            c = g * 2 + b
            wait_gather(gb, sg)

            @pl.when(g >= 1)
            def _():
                wait_scatter(sb, ss)

            scale_copy(gb, sb)

            @pl.when(c + 2 < N_CHUNKS)
            def _():
                start_gather(c + 2, gb, sg)

            start_scatter(c, sb, ss)
        return 0

    lax.fori_loop(0, N_CHUNKS // 2, step, 0)
    wait_scatter(s0, sems0)
    wait_scatter(s1, sems1)


@jax.jit
def _embed(idx2d, embeddings):
    mesh = plsc.VectorSubcoreMesh(core_axis_name="c", subcore_axis_name="s")
    run = pl.kernel(
        _body,
        out_type=jax.ShapeDtypeStruct((B_TOTAL, D), jnp.float32),
        mesh=mesh,
        scratch_types=[
            pltpu.VMEM((N_CHUNKS, CHUNK), jnp.int32),
            pltpu.VMEM((CHUNK, D), jnp.float32),
            pltpu.VMEM((CHUNK, D), jnp.float32),
            pltpu.VMEM((CHUNK, D), jnp.float32),
            pltpu.VMEM((CHUNK, D), jnp.float32),
            pltpu.SemaphoreType.DMA,
            pltpu.SemaphoreType.DMA,
            pltpu.SemaphoreType.DMA,
            pltpu.SemaphoreType.DMA,
        ],
    )
    return run(idx2d, embeddings)


def kernel(inputs, embeddings):
    idx2d = inputs.astype(jnp.int32).reshape(B_TOTAL // CHUNK, CHUNK)
    out = _embed(idx2d, embeddings)
    return out.reshape(inputs.shape[0], inputs.shape[1], D)


# speedup vs baseline: 1.0002x; 1.0002x over previous
import jax
import jax.numpy as jnp
from jax import lax
from jax.experimental import pallas as pl
from jax.experimental.pallas import tpu as pltpu
from jax.experimental.pallas import tpu_sc as plsc

VOCAB = 100000
D = 128
B_TOTAL = 4096 * 200
SCALE = float(D) ** 0.5

NC, NS, L = 2, 16, 16
NW = NC * NS
ROWS_PER_W = B_TOTAL // NW
CHUNK = 128
N_CHUNKS = ROWS_PER_W // CHUNK


def _body(idx_hbm, table_hbm, out_hbm,
          idx_v, g0, g1, s0, s1, semg0, semg1, sems0, sems1):
    cid = lax.axis_index("c")
    sid = lax.axis_index("s")
    wid = sid * NC + cid

    pltpu.sync_copy(idx_hbm.at[pl.ds(wid * N_CHUNKS, N_CHUNKS)], idx_v)

    out_base = pl.multiple_of(wid * ROWS_PER_W, CHUNK)

    def start_gather(c, buf, sem):
        pltpu.async_copy(table_hbm.at[idx_v.at[c]], buf, sem)

    def wait_gather(buf, sem):
        pltpu.make_async_copy(table_hbm.at[pl.ds(0, CHUNK)], buf, sem).wait()

    def start_scatter(c, buf, sem):
        pltpu.async_copy(buf, out_hbm.at[pl.ds(out_base + c * CHUNK, CHUNK)], sem)

    def wait_scatter(buf, sem):
        pltpu.make_async_copy(buf, out_hbm.at[pl.ds(0, CHUNK)], sem).wait()

    def scale_copy(src, dst):
        def row(r, _):
            for l in range(D // L):
                sl = pl.ds(l * L, L)
                dst[r, sl] = src[r, sl] * SCALE
            return 0
        lax.fori_loop(0, CHUNK, row, 0, unroll=4)

    bufs = ((g0, semg0, s0, sems0), (g1, semg1, s1, sems1))

    start_gather(0, g0, semg0)
    start_gather(1, g1, semg1)

    for b, (gb, sg, sb, ss) in enumerate(bufs):
        wait_gather(gb, sg)
        scale_copy(gb, sb)
        start_gather(2 + b, gb, sg)
        start_scatter(b, sb, ss)

    def step(g, _):
        for b, (gb, sg, sb, ss) in enumerate(bufs):
            c = g * 2 + b
            wait_gather(gb, sg)
            wait_scatter(sb, ss)
            scale_copy(gb, sb)
            start_gather(c + 2, gb, sg)
            start_scatter(c, sb, ss)
        return 0

    lax.fori_loop(1, N_CHUNKS // 2 - 1, step, 0)

    for b, (gb, sg, sb, ss) in enumerate(bufs):
        c = N_CHUNKS - 2 + b
        wait_gather(gb, sg)
        wait_scatter(sb, ss)
        scale_copy(gb, sb)
        start_scatter(c, sb, ss)

    wait_scatter(s0, sems0)
    wait_scatter(s1, sems1)


@jax.jit
def _embed(idx2d, embeddings):
    mesh = plsc.VectorSubcoreMesh(core_axis_name="c", subcore_axis_name="s")
    run = pl.kernel(
        _body,
        out_type=jax.ShapeDtypeStruct((B_TOTAL, D), jnp.float32),
        mesh=mesh,
        scratch_types=[
            pltpu.VMEM((N_CHUNKS, CHUNK), jnp.int32),
            pltpu.VMEM((CHUNK, D), jnp.float32),
            pltpu.VMEM((CHUNK, D), jnp.float32),
            pltpu.VMEM((CHUNK, D), jnp.float32),
            pltpu.VMEM((CHUNK, D), jnp.float32),
            pltpu.SemaphoreType.DMA,
            pltpu.SemaphoreType.DMA,
            pltpu.SemaphoreType.DMA,
            pltpu.SemaphoreType.DMA,
        ],
    )
    return run(idx2d, embeddings)


def kernel(inputs, embeddings):
    idx2d = inputs.astype(jnp.int32).reshape(B_TOTAL // CHUNK, CHUNK)
    out = _embed(idx2d, embeddings)
    return out.reshape(inputs.shape[0], inputs.shape[1], D)


# speedup vs baseline: 2.9657x; 2.9652x over previous
import jax
import jax.numpy as jnp
from jax import lax
from jax.experimental import pallas as pl
from jax.experimental.pallas import tpu as pltpu
from jax.experimental.pallas import tpu_sc as plsc

VOCAB = 100000
D = 128
B_TOTAL = 4096 * 200
SCALE = float(D) ** 0.5

NC, NS, L = 2, 16, 16
NW = NC * NS
ROWS_PER_W = B_TOTAL // NW
IROW = 128
CHUNK = 256
GPC = CHUNK // IROW
N_IDX_ROWS = ROWS_PER_W // IROW
N_CHUNKS = ROWS_PER_W // CHUNK


def _body(idx_hbm, table_hbm, out_hbm, idx_v, buf0, buf1, sem0, sem1):
    cid = lax.axis_index("c")
    sid = lax.axis_index("s")
    wid = sid * NC + cid

    pltpu.sync_copy(idx_hbm.at[pl.ds(wid * N_IDX_ROWS, N_IDX_ROWS)], idx_v)

    out_base = pl.multiple_of(wid * ROWS_PER_W, CHUNK)

    def start_gather(c, buf, sem):
        for j in range(GPC):
            pltpu.async_copy(table_hbm.at[idx_v.at[c * GPC + j]],
                             buf.at[pl.ds(j * IROW, IROW)], sem)

    def wait_gather(buf, sem):
        pltpu.make_async_copy(table_hbm.at[pl.ds(0, CHUNK)], buf, sem).wait()

    def scale_rows(buf):
        def row(r, _):
            for l in range(D // L):
                sl = pl.ds(l * L, L)
                buf[r, sl] = buf[r, sl] * SCALE
            return 0
        lax.fori_loop(0, CHUNK, row, 0, unroll=4)

    def flush(c, buf):
        scale_rows(buf)
        pltpu.sync_copy(buf, out_hbm.at[pl.ds(out_base + c * CHUNK, CHUNK)])

    start_gather(0, buf0, sem0)

    def step(g, _):
        c0 = g * 2
        start_gather(c0 + 1, buf1, sem1)
        wait_gather(buf0, sem0)
        flush(c0, buf0)

        @pl.when(c0 + 2 < N_CHUNKS)
        def _():
            start_gather(c0 + 2, buf0, sem0)

        wait_gather(buf1, sem1)
        flush(c0 + 1, buf1)
        return 0

    lax.fori_loop(0, N_CHUNKS // 2, step, 0)


@jax.jit
def _embed(idx2d, embeddings):
    mesh = plsc.VectorSubcoreMesh(core_axis_name="c", subcore_axis_name="s")
    run = pl.kernel(
        _body,
        out_type=jax.ShapeDtypeStruct((B_TOTAL, D), jnp.float32),
        mesh=mesh,
        scratch_types=[
            pltpu.VMEM((N_IDX_ROWS, IROW), jnp.int32),
            pltpu.VMEM((CHUNK, D), jnp.float32),
            pltpu.VMEM((CHUNK, D), jnp.float32),
            pltpu.SemaphoreType.DMA,
            pltpu.SemaphoreType.DMA,
        ],
    )
    return run(idx2d, embeddings)


def kernel(inputs, embeddings):
    idx2d = inputs.astype(jnp.int32).reshape(B_TOTAL // IROW, IROW)
    out = _embed(idx2d, embeddings)
    return out.reshape(inputs.shape[0], inputs.shape[1], D)


# speedup vs baseline: 2.9754x; 1.0033x over previous
import jax
import jax.numpy as jnp
from jax import lax
from jax.experimental import pallas as pl
from jax.experimental.pallas import tpu as pltpu
from jax.experimental.pallas import tpu_sc as plsc

VOCAB = 100000
D = 128
B_TOTAL = 4096 * 200
SCALE = float(D) ** 0.5

NC, NS, L = 2, 16, 16
NW = NC * NS
ROWS_PER_W = B_TOTAL // NW
IROW = 128
CHUNK = 256
GPC = CHUNK // IROW
N_IDX_ROWS = ROWS_PER_W // IROW
N_CHUNKS = ROWS_PER_W // CHUNK


def _body(idx_hbm, table_hbm, out_hbm, idx_v, buf0, buf1, buf2,
          sem0, sem1, sem2, semo0, semo1, semo2):
    cid = lax.axis_index("c")
    sid = lax.axis_index("s")
    wid = sid * NC + cid

    pltpu.sync_copy(idx_hbm.at[pl.ds(wid * N_IDX_ROWS, N_IDX_ROWS)], idx_v)

    out_base = pl.multiple_of(wid * ROWS_PER_W, CHUNK)

    def start_gather(c, buf, sem):
        for j in range(GPC):
            pltpu.async_copy(table_hbm.at[idx_v.at[c * GPC + j]],
                             buf.at[pl.ds(j * IROW, IROW)], sem)

    def wait_gather(buf, sem):
        pltpu.make_async_copy(table_hbm.at[pl.ds(0, CHUNK)], buf, sem).wait()

    def scale_rows(buf):
        def row(r, _):
            for l in range(D // L):
                sl = pl.ds(l * L, L)
                buf[r, sl] = buf[r, sl] * SCALE
            return 0
        lax.fori_loop(0, CHUNK, row, 0, unroll=4)

    def start_scatter(c, buf, sem):
        pltpu.async_copy(buf, out_hbm.at[pl.ds(out_base + c * CHUNK, CHUNK)], sem)

    def wait_scatter(buf, sem):
        pltpu.make_async_copy(buf, out_hbm.at[pl.ds(0, CHUNK)], sem).wait()

    def phase(c, bufs_sems, first, last):
        gb, sg, ss = bufs_sems[c % 3]
        wait_gather(gb, sg)
        scale_rows(gb)
        start_scatter(c, gb, ss)
        if not last:
            g2, sg2, ss2 = bufs_sems[(c + 2) % 3]
            if not first:
                wait_scatter(g2, ss2)
            start_gather(c + 2, g2, sg2)

    bs = ((buf0, sem0, semo0), (buf1, sem1, semo1), (buf2, sem2, semo2))

    start_gather(0, buf0, sem0)
    start_gather(1, buf1, sem1)
    phase(0, bs, first=True, last=False)
    phase(1, bs, first=False, last=False)

    def step(g, _):
        c0 = g * 3 + 2
        for k in range(3):
            gb, sg, ss = bs[(2 + k) % 3]
            g2, sg2, ss2 = bs[(2 + k + 2) % 3]
            c = c0 + k
            wait_gather(gb, sg)
            scale_rows(gb)
            start_scatter(c, gb, ss)
            wait_scatter(g2, ss2)
            start_gather(c + 2, g2, sg2)
        return 0

    lax.fori_loop(0, (N_CHUNKS - 4) // 3, step, 0)
    phase(N_CHUNKS - 2, bs, first=False, last=True)
    phase(N_CHUNKS - 1, bs, first=False, last=True)
    for gb, sg, ss in bs:
        wait_scatter(gb, ss)


@jax.jit
def _embed(idx2d, embeddings):
    mesh = plsc.VectorSubcoreMesh(core_axis_name="c", subcore_axis_name="s")
    run = pl.kernel(
        _body,
        out_type=jax.ShapeDtypeStruct((B_TOTAL, D), jnp.float32),
        mesh=mesh,
        scratch_types=[
            pltpu.VMEM((N_IDX_ROWS, IROW), jnp.int32),
            pltpu.VMEM((CHUNK, D), jnp.float32),
            pltpu.VMEM((CHUNK, D), jnp.float32),
            pltpu.VMEM((CHUNK, D), jnp.float32),
            pltpu.SemaphoreType.DMA,
            pltpu.SemaphoreType.DMA,
            pltpu.SemaphoreType.DMA,
            pltpu.SemaphoreType.DMA,
            pltpu.SemaphoreType.DMA,
            pltpu.SemaphoreType.DMA,
        ],
    )
    return run(idx2d, embeddings)


def kernel(inputs, embeddings):
    idx2d = inputs.astype(jnp.int32).reshape(B_TOTAL // IROW, IROW)
    out = _embed(idx2d, embeddings)
    return out.reshape(inputs.shape[0], inputs.shape[1], D)
